# Initial kernel scaffold; baseline (speedup 1.0000x reference)
import jax
import jax.numpy as jnp
from jax.experimental import pallas as pl


def kernel(scaler_logits, width_params, min_val, max_val):
    raise NotImplementedError("write your pallas kernel here")


# speedup vs baseline: 2.4791x; 2.4791x over previous
import functools

import jax
import jax.numpy as jnp
from jax import lax
from jax.experimental import pallas as pl
from jax.experimental.pallas import tpu as pltpu
from jax.experimental.pallas import tpu_sc as plsc

_B = 8
_C = 64
_H = 224
_W = 224
_PIX = _H * _W
_NC = 2
_NS = 16
_NW = _NC * _NS
_WPB = 4
_QPIX = _PIX // _WPB
_CHUNK = 896
_NCHUNK = _QPIX // _CHUNK
_G = _CHUNK // 16
_BISECT = 12
_POLISH = 2


def _centers_body(w_ref, mn_ref, mx_ref, o_ref):
    w = w_ref[...]
    mn = mn_ref[...]
    mx = mx_ref[...]
    sp = jnp.maximum(w, 0.0) + jnp.log1p(jnp.exp(-jnp.abs(w)))
    wn = sp / jnp.sum(sp) * (mx - mn)
    row = lax.broadcasted_iota(jnp.int32, (_C, _C), 0)
    col = lax.broadcasted_iota(jnp.int32, (_C, _C), 1)
    tri = (row <= col).astype(jnp.float32)
    cs = jnp.dot(wn, tri, preferred_element_type=jnp.float32)
    o_ref[...] = mn + cs


def _bin_centers(width_params, min_val, max_val):
    w = width_params.reshape(1, _C)
    mn = min_val.reshape(1, 1)
    mx = max_val.reshape(1, 1)
    out = pl.pallas_call(
        _centers_body,
        out_shape=jax.ShapeDtypeStruct((1, _C), jnp.float32),
    )(w, mn, mx)
    return out.reshape(_C)


def _sc_body(x_hbm, ctr_hbm, out_hbm, xb, res, ctr):
    cid = lax.axis_index("c")
    sid = lax.axis_index("s")
    wid = sid * _NC + cid
    b = wid // _WPB
    qbase = (wid % _WPB) * _QPIX

    pltpu.sync_copy(ctr_hbm, ctr)

    def chunk_body(k, carry):
        off = pl.multiple_of(qbase + k * _CHUNK, 128)
        pltpu.sync_copy(x_hbm.at[b, :, pl.ds(off, _CHUNK)], xb)

        def group_body(j, carry2):
            sl = pl.ds(j * 16, 16)

            def max_body(c, ms):
                m0, m1, m2, m3 = ms
                c4 = c * 4
                return (jnp.maximum(m0, xb[c4, sl]),
                        jnp.maximum(m1, xb[c4 + 1, sl]),
                        jnp.maximum(m2, xb[c4 + 2, sl]),
                        jnp.maximum(m3, xb[c4 + 3, sl]))

            ninf = jnp.full((16,), -jnp.inf, jnp.float32)
            m0, m1, m2, m3 = lax.fori_loop(
                0, _C // 4, max_body, (ninf, ninf, ninf, ninf))
            m = jnp.maximum(jnp.maximum(m0, m1), jnp.maximum(m2, m3))

            lo = m - 1.0
            hi = m - (1.0 / _C)

            def bisect_body(i, bracket):
                blo, bhi = bracket
                mid = 0.5 * (blo + bhi)

                def s_body(c, accs):
                    a0, a1, a2, a3 = accs
                    c4 = c * 4
                    return (a0 + jnp.maximum(xb[c4, sl] - mid, 0.0),
                            a1 + jnp.maximum(xb[c4 + 1, sl] - mid, 0.0),
                            a2 + jnp.maximum(xb[c4 + 2, sl] - mid, 0.0),
                            a3 + jnp.maximum(xb[c4 + 3, sl] - mid, 0.0))

                z = jnp.zeros((16,), jnp.float32)
                a0, a1, a2, a3 = lax.fori_loop(0, _C // 4, s_body,
                                               (z, z, z, z))
                s = (a0 + a1) + (a2 + a3)
                g = s > 1.0
                return (jnp.where(g, mid, blo), jnp.where(g, bhi, mid))

            lo, hi = lax.fori_loop(0, _BISECT, bisect_body, (lo, hi))
            t = lo

            for _ in range(_POLISH):
                def sc_body(c, accs):
                    a0, a1, a2, a3, n0, n1, n2, n3 = accs
                    c4 = c * 4
                    d0 = xb[c4, sl] - t
                    d1 = xb[c4 + 1, sl] - t
                    d2 = xb[c4 + 2, sl] - t
                    d3 = xb[c4 + 3, sl] - t
                    one = jnp.float32(1.0)
                    zero = jnp.float32(0.0)
                    return (a0 + jnp.maximum(d0, 0.0),
                            a1 + jnp.maximum(d1, 0.0),
                            a2 + jnp.maximum(d2, 0.0),
                            a3 + jnp.maximum(d3, 0.0),
                            n0 + jnp.where(d0 > 0.0, one, zero),
                            n1 + jnp.where(d1 > 0.0, one, zero),
                            n2 + jnp.where(d2 > 0.0, one, zero),
                            n3 + jnp.where(d3 > 0.0, one, zero))

                z = jnp.zeros((16,), jnp.float32)
                a0, a1, a2, a3, n0, n1, n2, n3 = lax.fori_loop(
                    0, _C // 4, sc_body, (z, z, z, z, z, z, z, z))
                s = (a0 + a1) + (a2 + a3)
                cnt = (n0 + n1) + (n2 + n3)
                t = jnp.minimum(t + (s - 1.0) / jnp.maximum(cnt, 1.0), hi)

            z = jnp.zeros((16,), jnp.float32)
            accs = [z, z, z, z]
            for vb in range(_C // 16):
                cv = ctr[pl.ds(vb * 16, 16)]
                for k in range(16):
                    c = vb * 16 + k
                    accs[k % 4] = (accs[k % 4]
                                   + jnp.maximum(xb[c, sl] - t, 0.0) * cv[k])
            res[sl] = (accs[0] + accs[1]) + (accs[2] + accs[3])
            return carry2

        lax.fori_loop(0, _G, group_body, 0)
        obase = pl.multiple_of(b * _PIX + off, 8)
        pltpu.sync_copy(res, out_hbm.at[pl.ds(obase, _CHUNK)])
        return carry

    lax.fori_loop(0, _NCHUNK, chunk_body, 0)


_sc_sparsemax = functools.partial(
    pl.kernel,
    out_type=jax.ShapeDtypeStruct((_B * _PIX,), jnp.float32),
    mesh=plsc.VectorSubcoreMesh(core_axis_name="c", subcore_axis_name="s"),
    scratch_types=[
        pltpu.VMEM((_C, _CHUNK), jnp.float32),
        pltpu.VMEM((_CHUNK,), jnp.float32),
        pltpu.VMEM((_C,), jnp.float32),
    ],
)(_sc_body)


def kernel(scaler_logits, width_params, min_val, max_val):
    x = scaler_logits.reshape(_B, _C, _PIX)
    centers = _bin_centers(width_params, min_val, max_val)
    out = _sc_sparsemax(x, centers)
    return out.reshape(_B, 1, _H, _W)
